# baseline (speedup 1.0000x reference)
import functools

import jax
import jax.numpy as jnp
from jax import lax
from jax.experimental import pallas as pl
from jax.experimental.pallas import tpu as pltpu
from jax.experimental.pallas import tpu_sc as plsc

N = 10000
E = 320000
D_NODE = 128
D_EDGE = 16
LATENT = 128
OUT_D = 128
N_GRAPHS = 100
MAX_LV = 3

NUM_SC = 2
NUM_TILES = 16
NW = NUM_SC * NUM_TILES

CHUNK = 128
NCH = E // CHUNK
NCHT = 80
SCHUNK = 16
NSTAGE = NCHT // SCHUNK
ACC_N = N + 8
TROWS = 640
ZROWS = 80

_TILE_CHUNKS = [79] * 4 + [78] * (NW - 4)


def _pad_chunks(arr2d, fill):
    parts = []
    off = 0
    for w in range(NW):
        nkw = _TILE_CHUNKS[w]
        parts.append(arr2d[off:off + nkw])
        parts.append(jnp.full((NCHT - nkw, CHUNK), fill, jnp.int32))
        off += nkw
    return jnp.concatenate(parts, axis=0)


@functools.cache
def _mesh():
    return plsc.VectorSubcoreMesh(core_axis_name="c", subcore_axis_name="s")


def _make_gs_body(width):
    def body(x_hbm, src_hbm, dst_hbm, z_hbm, out_hbm,
             idx_s, idx_d, buf0, buf1, acc, g0, g1):
        c = lax.axis_index("c")
        t = lax.axis_index("s")
        w = c * NUM_TILES + t
        row0 = pl.multiple_of(w * NCHT, 8)

        base = t * TROWS
        nrows = jnp.minimum(N - base, TROWS)

        def zc(k, _):
            r = pl.multiple_of(base + k * ZROWS, 8)
            pltpu.sync_copy(z_hbm, acc.at[pl.ds(r, ZROWS)])
            return 0

        lax.fori_loop(0, nrows // ZROWS, zc, 0)
        plsc.subcore_barrier()

        def stage(st, _):
            srow = pl.multiple_of(row0 + st * SCHUNK, 8)
            pltpu.sync_copy(src_hbm.at[pl.ds(srow, SCHUNK)], idx_s)
            pltpu.sync_copy(dst_hbm.at[pl.ds(srow, SCHUNK)], idx_d)
            pltpu.async_copy(x_hbm.at[idx_s.at[0]], buf0, g0)

            def pair(jj, _):
                j0 = 2 * jj
                j1 = 2 * jj + 1
                pltpu.async_copy(x_hbm.at[idx_s.at[j1]], buf1, g1)
                pltpu.make_async_copy(
                    x_hbm.at[idx_s.at[j0]], buf0, g0).wait()
                pltpu.sync_copy(buf0, acc.at[idx_d.at[j0]], add=True)

                @pl.when(jj + 1 < SCHUNK // 2)
                def _():
                    pltpu.async_copy(x_hbm.at[idx_s.at[j0 + 2]], buf0, g0)

                pltpu.make_async_copy(
                    x_hbm.at[idx_s.at[j1]], buf1, g1).wait()
                pltpu.sync_copy(buf1, acc.at[idx_d.at[j1]], add=True)
                return 0

            lax.fori_loop(0, SCHUNK // 2, pair, 0)
            return 0

        lax.fori_loop(0, NSTAGE, stage, 0)
        plsc.subcore_barrier()

        def co(k, _):
            r = pl.multiple_of(base + k * ZROWS, 8)
            pltpu.sync_copy(acc.at[pl.ds(r, ZROWS)],
                            out_hbm.at[c, pl.ds(r, ZROWS)])
            return 0

        lax.fori_loop(0, nrows // ZROWS, co, 0)

    return body


@functools.cache
def _gs_kernel(width):
    return pl.kernel(
        _make_gs_body(width),
        out_type=jax.ShapeDtypeStruct((NUM_SC, N, width), jnp.float32),
        name=f"sc_segsum_{width}",
        mesh=_mesh(),
        scratch_types=[
            pltpu.VMEM((SCHUNK, CHUNK), jnp.int32),
            pltpu.VMEM((SCHUNK, CHUNK), jnp.int32),
            pltpu.VMEM((CHUNK, width), jnp.float32),
            pltpu.VMEM((CHUNK, width), jnp.float32),
            pltpu.VMEM_SHARED((ACC_N, width), jnp.float32),
            pltpu.SemaphoreType.DMA,
            pltpu.SemaphoreType.DMA,
        ],
    )


BLK = 1000
NBLK = N // BLK


EBLK = 8000
NEBLK = E // EBLK


def _tc0_body(ef_ref, we_ref, b_ref, emb_ref):
    emb_ref[...] = (
        jnp.dot(ef_ref[...], we_ref[...], preferred_element_type=jnp.float32)
        + b_ref[...])


def _tc0(edge_feat, W_e2l, b):
    return pl.pallas_call(
        _tc0_body,
        grid=(NEBLK,),
        in_specs=[
            pl.BlockSpec((EBLK, D_EDGE), lambda i: (i, 0)),
            pl.BlockSpec((D_EDGE, LATENT), lambda i: (0, 0)),
            pl.BlockSpec((1, LATENT), lambda i: (0, 0)),
        ],
        out_specs=pl.BlockSpec((EBLK, LATENT), lambda i: (i, 0)),
        out_shape=jax.ShapeDtypeStruct((E, LATENT), jnp.float32),
        compiler_params=pltpu.CompilerParams(
            dimension_semantics=("parallel",)),
    )(edge_feat, W_e2l, b)


def _tc1_body(nf_ref, p_ref, wn_ref, b_ref, msg_ref, cur_ref):
    p = p_ref[0] + p_ref[1]
    m = (jnp.dot(nf_ref[...], wn_ref[...], preferred_element_type=jnp.float32)
         + p + b_ref[...])
    msg_ref[...] = m
    cur_ref[...] = jnp.maximum(m, 0.0)


def _tc1(node_feat, pool, W_n2l, b):
    return pl.pallas_call(
        _tc1_body,
        grid=(NBLK,),
        in_specs=[
            pl.BlockSpec((BLK, D_NODE), lambda i: (i, 0)),
            pl.BlockSpec((NUM_SC, BLK, LATENT), lambda i: (0, i, 0)),
            pl.BlockSpec((D_NODE, LATENT), lambda i: (0, 0)),
            pl.BlockSpec((1, LATENT), lambda i: (0, 0)),
        ],
        out_specs=[
            pl.BlockSpec((BLK, LATENT), lambda i: (i, 0)),
            pl.BlockSpec((BLK, LATENT), lambda i: (i, 0)),
        ],
        out_shape=[
            jax.ShapeDtypeStruct((N, LATENT), jnp.float32),
            jax.ShapeDtypeStruct((N, LATENT), jnp.float32),
        ],
        compiler_params=pltpu.CompilerParams(
            dimension_semantics=("parallel",)),
    )(node_feat, pool, W_n2l, b)


def _tc2_body(p_ref, msg_ref, w_ref, b_ref, cur_ref):
    p = p_ref[0] + p_ref[1]
    m = (jnp.dot(p, w_ref[...], preferred_element_type=jnp.float32)
         + b_ref[...] + msg_ref[...])
    cur_ref[...] = jnp.maximum(m, 0.0)


def _tc2(parts, msg, W_conv, b):
    return pl.pallas_call(
        _tc2_body,
        grid=(NBLK,),
        in_specs=[
            pl.BlockSpec((NUM_SC, BLK, LATENT), lambda i: (0, i, 0)),
            pl.BlockSpec((BLK, LATENT), lambda i: (i, 0)),
            pl.BlockSpec((LATENT, LATENT), lambda i: (0, 0)),
            pl.BlockSpec((1, LATENT), lambda i: (0, 0)),
        ],
        out_specs=pl.BlockSpec((BLK, LATENT), lambda i: (i, 0)),
        out_shape=jax.ShapeDtypeStruct((N, LATENT), jnp.float32),
        compiler_params=pltpu.CompilerParams(
            dimension_semantics=("parallel",)),
    )(parts, msg, W_conv, b)


def _tc3_body(cur_ref, gid_ref, w_ref, b_ref, relu_ref, y_ref):
    i = pl.program_id(0)
    r = jnp.maximum(
        jnp.dot(cur_ref[...], w_ref[...], preferred_element_type=jnp.float32)
        + b_ref[...], 0.0)
    relu_ref[...] = r
    gid = gid_ref[0]
    onehot = (lax.broadcasted_iota(jnp.int32, (N_GRAPHS, BLK), 0)
              == gid).astype(jnp.float32)
    part = jnp.dot(onehot, r, preferred_element_type=jnp.float32)

    @pl.when(i == 0)
    def _():
        y_ref[...] = jnp.zeros_like(y_ref)

    acc = y_ref[...] + part
    y_ref[...] = jnp.where(i == NBLK - 1, jnp.maximum(acc, 0.0), acc)


def _tc3(cur, gids3d, W_out, b):
    return pl.pallas_call(
        _tc3_body,
        grid=(NBLK,),
        in_specs=[
            pl.BlockSpec((BLK, LATENT), lambda i: (i, 0)),
            pl.BlockSpec((1, 1, BLK), lambda i: (i, 0, 0)),
            pl.BlockSpec((LATENT, OUT_D), lambda i: (0, 0)),
            pl.BlockSpec((1, OUT_D), lambda i: (0, 0)),
        ],
        out_specs=[
            pl.BlockSpec((BLK, OUT_D), lambda i: (i, 0)),
            pl.BlockSpec((N_GRAPHS, OUT_D), lambda i: (0, 0)),
        ],
        out_shape=[
            jax.ShapeDtypeStruct((N, OUT_D), jnp.float32),
            jax.ShapeDtypeStruct((N_GRAPHS, OUT_D), jnp.float32),
        ],
        compiler_params=pltpu.CompilerParams(
            dimension_semantics=("arbitrary",)),
    )(cur, gids3d, W_out, b)


def kernel(node_feat, edge_feat, edge_index, graph_ids,
           W_n2l, b_n2l, W_e2l, b_e2l, W_conv, b_conv, W_out, b_out):
    src2d = _pad_chunks(edge_index[0].reshape(NCH, CHUNK), 0)
    dst2d = _pad_chunks(edge_index[1].reshape(NCH, CHUNK), N)
    eidx2d = _pad_chunks(jnp.arange(E, dtype=jnp.int32).reshape(NCH, CHUNK), 0)
    gids3d = graph_ids.reshape(NBLK, 1, BLK)
    zrows = jnp.zeros((ZROWS, LATENT), jnp.float32)

    edge_emb = _tc0(edge_feat, W_e2l, b_e2l.reshape(1, LATENT))
    pool = _gs_kernel(LATENT)(edge_emb, eidx2d, dst2d, zrows)
    msg, cur = _tc1(node_feat, pool, W_n2l, b_n2l.reshape(1, LATENT))
    for _ in range(MAX_LV):
        parts = _gs_kernel(LATENT)(cur, src2d, dst2d, zrows)
        cur = _tc2(parts, msg, W_conv, b_conv.reshape(1, LATENT))
    reluact, y = _tc3(cur, gids3d, W_out, b_out.reshape(1, OUT_D))
    return (reluact, y)


# speedup vs baseline: 1.0006x; 1.0006x over previous
import functools

import jax
import jax.numpy as jnp
from jax import lax
from jax.experimental import pallas as pl
from jax.experimental.pallas import tpu as pltpu
from jax.experimental.pallas import tpu_sc as plsc

N = 10000
E = 320000
D_NODE = 128
D_EDGE = 16
LATENT = 128
OUT_D = 128
N_GRAPHS = 100
MAX_LV = 3

NUM_SC = 2
NUM_TILES = 16
NW = NUM_SC * NUM_TILES

CHUNK = 128
NCH = E // CHUNK
NCHT = 80
SCHUNK = 16
NSTAGE = NCHT // SCHUNK
ACC_N = N + 8
TROWS = 640
ZROWS = 80

_TILE_CHUNKS = [79] * 4 + [78] * (NW - 4)


def _pad_chunks(arr2d, fill):
    parts = []
    off = 0
    for w in range(NW):
        nkw = _TILE_CHUNKS[w]
        parts.append(arr2d[off:off + nkw])
        parts.append(jnp.full((NCHT - nkw, CHUNK), fill, jnp.int32))
        off += nkw
    return jnp.concatenate(parts, axis=0)


@functools.cache
def _mesh():
    return plsc.VectorSubcoreMesh(core_axis_name="c", subcore_axis_name="s")


def _make_gs_body(width):
    def body(x_hbm, src_hbm, dst_hbm, z_hbm, out_hbm,
             idx_s, idx_d, buf0, buf1, acc, g0, g1):
        c = lax.axis_index("c")
        t = lax.axis_index("s")
        w = c * NUM_TILES + t
        row0 = pl.multiple_of(w * NCHT, 8)
        nreal = jnp.where(w < 4, 79, 78)

        base = t * TROWS
        nrows = jnp.minimum(N - base, TROWS)

        def zc(k, _):
            r = pl.multiple_of(base + k * ZROWS, 8)
            pltpu.sync_copy(z_hbm, acc.at[pl.ds(r, ZROWS)])
            return 0

        lax.fori_loop(0, nrows // ZROWS, zc, 0)
        plsc.subcore_barrier()

        def stage(st, _):
            srow = pl.multiple_of(row0 + st * SCHUNK, 8)
            pltpu.sync_copy(src_hbm.at[pl.ds(srow, SCHUNK)], idx_s)
            pltpu.sync_copy(dst_hbm.at[pl.ds(srow, SCHUNK)], idx_d)
            pltpu.async_copy(x_hbm.at[idx_s.at[0]], buf0, g0)

            def pair(jj, _):
                j0 = 2 * jj
                j1 = 2 * jj + 1
                pltpu.async_copy(x_hbm.at[idx_s.at[j1]], buf1, g1)
                pltpu.make_async_copy(
                    x_hbm.at[idx_s.at[j0]], buf0, g0).wait()

                @pl.when(st * SCHUNK + j0 < nreal)
                def _():
                    pltpu.sync_copy(buf0, acc.at[idx_d.at[j0]], add=True)

                @pl.when(jj + 1 < SCHUNK // 2)
                def _():
                    pltpu.async_copy(x_hbm.at[idx_s.at[j0 + 2]], buf0, g0)

                pltpu.make_async_copy(
                    x_hbm.at[idx_s.at[j1]], buf1, g1).wait()

                @pl.when(st * SCHUNK + j1 < nreal)
                def _():
                    pltpu.sync_copy(buf1, acc.at[idx_d.at[j1]], add=True)

                return 0

            lax.fori_loop(0, SCHUNK // 2, pair, 0)
            return 0

        lax.fori_loop(0, NSTAGE, stage, 0)
        plsc.subcore_barrier()

        def co(k, _):
            r = pl.multiple_of(base + k * ZROWS, 8)
            pltpu.sync_copy(acc.at[pl.ds(r, ZROWS)],
                            out_hbm.at[c, pl.ds(r, ZROWS)])
            return 0

        lax.fori_loop(0, nrows // ZROWS, co, 0)

    return body


@functools.cache
def _gs_kernel(width):
    return pl.kernel(
        _make_gs_body(width),
        out_type=jax.ShapeDtypeStruct((NUM_SC, N, width), jnp.float32),
        name=f"sc_segsum_{width}",
        mesh=_mesh(),
        scratch_types=[
            pltpu.VMEM((SCHUNK, CHUNK), jnp.int32),
            pltpu.VMEM((SCHUNK, CHUNK), jnp.int32),
            pltpu.VMEM((CHUNK, width), jnp.float32),
            pltpu.VMEM((CHUNK, width), jnp.float32),
            pltpu.VMEM_SHARED((ACC_N, width), jnp.float32),
            pltpu.SemaphoreType.DMA,
            pltpu.SemaphoreType.DMA,
        ],
    )


BLK = 1000
NBLK = N // BLK


EBLK = 8000
NEBLK = E // EBLK


def _tc0_body(ef_ref, we_ref, b_ref, emb_ref):
    emb_ref[...] = (
        jnp.dot(ef_ref[...], we_ref[...], preferred_element_type=jnp.float32)
        + b_ref[...])


def _tc0(edge_feat, W_e2l, b):
    return pl.pallas_call(
        _tc0_body,
        grid=(NEBLK,),
        in_specs=[
            pl.BlockSpec((EBLK, D_EDGE), lambda i: (i, 0)),
            pl.BlockSpec((D_EDGE, LATENT), lambda i: (0, 0)),
            pl.BlockSpec((1, LATENT), lambda i: (0, 0)),
        ],
        out_specs=pl.BlockSpec((EBLK, LATENT), lambda i: (i, 0)),
        out_shape=jax.ShapeDtypeStruct((E, LATENT), jnp.float32),
        compiler_params=pltpu.CompilerParams(
            dimension_semantics=("parallel",)),
    )(edge_feat, W_e2l, b)


def _tc1_body(nf_ref, p_ref, wn_ref, b_ref, msg_ref, cur_ref):
    p = p_ref[0] + p_ref[1]
    m = (jnp.dot(nf_ref[...], wn_ref[...], preferred_element_type=jnp.float32)
         + p + b_ref[...])
    msg_ref[...] = m
    cur_ref[...] = jnp.maximum(m, 0.0)


def _tc1(node_feat, pool, W_n2l, b):
    return pl.pallas_call(
        _tc1_body,
        grid=(NBLK,),
        in_specs=[
            pl.BlockSpec((BLK, D_NODE), lambda i: (i, 0)),
            pl.BlockSpec((NUM_SC, BLK, LATENT), lambda i: (0, i, 0)),
            pl.BlockSpec((D_NODE, LATENT), lambda i: (0, 0)),
            pl.BlockSpec((1, LATENT), lambda i: (0, 0)),
        ],
        out_specs=[
            pl.BlockSpec((BLK, LATENT), lambda i: (i, 0)),
            pl.BlockSpec((BLK, LATENT), lambda i: (i, 0)),
        ],
        out_shape=[
            jax.ShapeDtypeStruct((N, LATENT), jnp.float32),
            jax.ShapeDtypeStruct((N, LATENT), jnp.float32),
        ],
        compiler_params=pltpu.CompilerParams(
            dimension_semantics=("parallel",)),
    )(node_feat, pool, W_n2l, b)


def _tc2_body(p_ref, msg_ref, w_ref, b_ref, cur_ref):
    p = p_ref[0] + p_ref[1]
    m = (jnp.dot(p, w_ref[...], preferred_element_type=jnp.float32)
         + b_ref[...] + msg_ref[...])
    cur_ref[...] = jnp.maximum(m, 0.0)


def _tc2(parts, msg, W_conv, b):
    return pl.pallas_call(
        _tc2_body,
        grid=(NBLK,),
        in_specs=[
            pl.BlockSpec((NUM_SC, BLK, LATENT), lambda i: (0, i, 0)),
            pl.BlockSpec((BLK, LATENT), lambda i: (i, 0)),
            pl.BlockSpec((LATENT, LATENT), lambda i: (0, 0)),
            pl.BlockSpec((1, LATENT), lambda i: (0, 0)),
        ],
        out_specs=pl.BlockSpec((BLK, LATENT), lambda i: (i, 0)),
        out_shape=jax.ShapeDtypeStruct((N, LATENT), jnp.float32),
        compiler_params=pltpu.CompilerParams(
            dimension_semantics=("parallel",)),
    )(parts, msg, W_conv, b)


def _tc3_body(cur_ref, gid_ref, w_ref, b_ref, relu_ref, y_ref):
    i = pl.program_id(0)
    r = jnp.maximum(
        jnp.dot(cur_ref[...], w_ref[...], preferred_element_type=jnp.float32)
        + b_ref[...], 0.0)
    relu_ref[...] = r
    gid = gid_ref[0]
    onehot = (lax.broadcasted_iota(jnp.int32, (N_GRAPHS, BLK), 0)
              == gid).astype(jnp.float32)
    part = jnp.dot(onehot, r, preferred_element_type=jnp.float32)

    @pl.when(i == 0)
    def _():
        y_ref[...] = jnp.zeros_like(y_ref)

    acc = y_ref[...] + part
    y_ref[...] = jnp.where(i == NBLK - 1, jnp.maximum(acc, 0.0), acc)


def _tc3(cur, gids3d, W_out, b):
    return pl.pallas_call(
        _tc3_body,
        grid=(NBLK,),
        in_specs=[
            pl.BlockSpec((BLK, LATENT), lambda i: (i, 0)),
            pl.BlockSpec((1, 1, BLK), lambda i: (i, 0, 0)),
            pl.BlockSpec((LATENT, OUT_D), lambda i: (0, 0)),
            pl.BlockSpec((1, OUT_D), lambda i: (0, 0)),
        ],
        out_specs=[
            pl.BlockSpec((BLK, OUT_D), lambda i: (i, 0)),
            pl.BlockSpec((N_GRAPHS, OUT_D), lambda i: (0, 0)),
        ],
        out_shape=[
            jax.ShapeDtypeStruct((N, OUT_D), jnp.float32),
            jax.ShapeDtypeStruct((N_GRAPHS, OUT_D), jnp.float32),
        ],
        compiler_params=pltpu.CompilerParams(
            dimension_semantics=("arbitrary",)),
    )(cur, gids3d, W_out, b)


def kernel(node_feat, edge_feat, edge_index, graph_ids,
           W_n2l, b_n2l, W_e2l, b_e2l, W_conv, b_conv, W_out, b_out):
    src2d = _pad_chunks(edge_index[0].reshape(NCH, CHUNK), 0)
    dst2d = _pad_chunks(edge_index[1].reshape(NCH, CHUNK), N)
    eidx2d = _pad_chunks(jnp.arange(E, dtype=jnp.int32).reshape(NCH, CHUNK), 0)
    gids3d = graph_ids.reshape(NBLK, 1, BLK)
    zrows = jnp.zeros((ZROWS, LATENT), jnp.float32)

    edge_emb = _tc0(edge_feat, W_e2l, b_e2l.reshape(1, LATENT))
    pool = _gs_kernel(LATENT)(edge_emb, eidx2d, dst2d, zrows)
    msg, cur = _tc1(node_feat, pool, W_n2l, b_n2l.reshape(1, LATENT))
    for _ in range(MAX_LV):
        parts = _gs_kernel(LATENT)(cur, src2d, dst2d, zrows)
        cur = _tc2(parts, msg, W_conv, b_conv.reshape(1, LATENT))
    reluact, y = _tc3(cur, gids3d, W_out, b_out.reshape(1, OUT_D))
    return (reluact, y)


# speedup vs baseline: 2.5879x; 2.5863x over previous
import functools

import jax
import jax.numpy as jnp
from jax import lax
from jax.experimental import pallas as pl
from jax.experimental.pallas import tpu as pltpu
from jax.experimental.pallas import tpu_sc as plsc

N = 10000
E = 320000
D_NODE = 128
D_EDGE = 16
LATENT = 128
OUT_D = 128
N_GRAPHS = 100
MAX_LV = 3

NUM_SC = 2
NUM_TILES = 16
NW = NUM_SC * NUM_TILES

CHUNK = 125
EPT = E // NW
NCHT = EPT // CHUNK
SCHUNK = 16
NSTAGE = NCHT // SCHUNK
ACC_N = N
TROWS = 640
ZROWS = 80


@functools.cache
def _mesh():
    return plsc.VectorSubcoreMesh(core_axis_name="c", subcore_axis_name="s")


def _make_gs_body(width):
    def body(x_hbm, src_hbm, dst_hbm, z_hbm, out_hbm,
             idx_s, idx_d, buf0, buf1, acc, g0, g1):
        c = lax.axis_index("c")
        t = lax.axis_index("s")
        w = c * NUM_TILES + t
        row0 = pl.multiple_of(w * NCHT, 8)

        base = t * TROWS
        nrows = jnp.minimum(N - base, TROWS)

        def zc(k, _):
            r = pl.multiple_of(base + k * ZROWS, 8)
            pltpu.sync_copy(z_hbm, acc.at[pl.ds(r, ZROWS)])
            return 0

        lax.fori_loop(0, nrows // ZROWS, zc, 0)
        plsc.subcore_barrier()

        def stage(st, _):
            srow = pl.multiple_of(row0 + st * SCHUNK, 8)
            pltpu.sync_copy(src_hbm.at[pl.ds(srow, SCHUNK)], idx_s)
            pltpu.sync_copy(dst_hbm.at[pl.ds(srow, SCHUNK)], idx_d)
            pltpu.async_copy(x_hbm.at[idx_s.at[0]], buf0, g0)

            def pair(jj, _):
                j0 = 2 * jj
                j1 = 2 * jj + 1
                pltpu.async_copy(x_hbm.at[idx_s.at[j1]], buf1, g1)
                pltpu.make_async_copy(
                    x_hbm.at[idx_s.at[j0]], buf0, g0).wait()
                pltpu.sync_copy(buf0, acc.at[idx_d.at[j0]], add=True)

                @pl.when(jj + 1 < SCHUNK // 2)
                def _():
                    pltpu.async_copy(x_hbm.at[idx_s.at[j0 + 2]], buf0, g0)

                pltpu.make_async_copy(
                    x_hbm.at[idx_s.at[j1]], buf1, g1).wait()
                pltpu.sync_copy(buf1, acc.at[idx_d.at[j1]], add=True)
                return 0

            lax.fori_loop(0, SCHUNK // 2, pair, 0)
            return 0

        lax.fori_loop(0, NSTAGE, stage, 0)
        plsc.subcore_barrier()

        def co(k, _):
            r = pl.multiple_of(base + k * ZROWS, 8)
            pltpu.sync_copy(acc.at[pl.ds(r, ZROWS)],
                            out_hbm.at[c, pl.ds(r, ZROWS)])
            return 0

        lax.fori_loop(0, nrows // ZROWS, co, 0)

    return body


@functools.cache
def _gs_kernel(width):
    return pl.kernel(
        _make_gs_body(width),
        out_type=jax.ShapeDtypeStruct((NUM_SC, N, width), jnp.float32),
        name=f"sc_segsum_{width}",
        mesh=_mesh(),
        scratch_types=[
            pltpu.VMEM((SCHUNK, CHUNK), jnp.int32),
            pltpu.VMEM((SCHUNK, CHUNK), jnp.int32),
            pltpu.VMEM((CHUNK, width), jnp.float32),
            pltpu.VMEM((CHUNK, width), jnp.float32),
            pltpu.VMEM_SHARED((ACC_N, width), jnp.float32),
            pltpu.SemaphoreType.DMA,
            pltpu.SemaphoreType.DMA,
        ],
    )


BLK = 1000
NBLK = N // BLK


EBLK = 8000
NEBLK = E // EBLK


def _tc0_body(ef_ref, we_ref, b_ref, emb_ref):
    emb_ref[...] = (
        jnp.dot(ef_ref[...], we_ref[...], preferred_element_type=jnp.float32)
        + b_ref[...])


def _tc0(edge_feat, W_e2l, b):
    return pl.pallas_call(
        _tc0_body,
        grid=(NEBLK,),
        in_specs=[
            pl.BlockSpec((EBLK, D_EDGE), lambda i: (i, 0)),
            pl.BlockSpec((D_EDGE, LATENT), lambda i: (0, 0)),
            pl.BlockSpec((1, LATENT), lambda i: (0, 0)),
        ],
        out_specs=pl.BlockSpec((EBLK, LATENT), lambda i: (i, 0)),
        out_shape=jax.ShapeDtypeStruct((E, LATENT), jnp.float32),
        compiler_params=pltpu.CompilerParams(
            dimension_semantics=("parallel",)),
    )(edge_feat, W_e2l, b)


def _tc1_body(nf_ref, p_ref, wn_ref, b_ref, msg_ref, cur_ref):
    p = p_ref[0] + p_ref[1]
    m = (jnp.dot(nf_ref[...], wn_ref[...], preferred_element_type=jnp.float32)
         + p + b_ref[...])
    msg_ref[...] = m
    cur_ref[...] = jnp.maximum(m, 0.0)


def _tc1(node_feat, pool, W_n2l, b):
    return pl.pallas_call(
        _tc1_body,
        grid=(NBLK,),
        in_specs=[
            pl.BlockSpec((BLK, D_NODE), lambda i: (i, 0)),
            pl.BlockSpec((NUM_SC, BLK, LATENT), lambda i: (0, i, 0)),
            pl.BlockSpec((D_NODE, LATENT), lambda i: (0, 0)),
            pl.BlockSpec((1, LATENT), lambda i: (0, 0)),
        ],
        out_specs=[
            pl.BlockSpec((BLK, LATENT), lambda i: (i, 0)),
            pl.BlockSpec((BLK, LATENT), lambda i: (i, 0)),
        ],
        out_shape=[
            jax.ShapeDtypeStruct((N, LATENT), jnp.float32),
            jax.ShapeDtypeStruct((N, LATENT), jnp.float32),
        ],
        compiler_params=pltpu.CompilerParams(
            dimension_semantics=("parallel",)),
    )(node_feat, pool, W_n2l, b)


def _tc2_body(p_ref, msg_ref, w_ref, b_ref, cur_ref):
    p = p_ref[0] + p_ref[1]
    m = (jnp.dot(p, w_ref[...], preferred_element_type=jnp.float32)
         + b_ref[...] + msg_ref[...])
    cur_ref[...] = jnp.maximum(m, 0.0)


def _tc2(parts, msg, W_conv, b):
    return pl.pallas_call(
        _tc2_body,
        grid=(NBLK,),
        in_specs=[
            pl.BlockSpec((NUM_SC, BLK, LATENT), lambda i: (0, i, 0)),
            pl.BlockSpec((BLK, LATENT), lambda i: (i, 0)),
            pl.BlockSpec((LATENT, LATENT), lambda i: (0, 0)),
            pl.BlockSpec((1, LATENT), lambda i: (0, 0)),
        ],
        out_specs=pl.BlockSpec((BLK, LATENT), lambda i: (i, 0)),
        out_shape=jax.ShapeDtypeStruct((N, LATENT), jnp.float32),
        compiler_params=pltpu.CompilerParams(
            dimension_semantics=("parallel",)),
    )(parts, msg, W_conv, b)


def _tc3_body(cur_ref, gid_ref, w_ref, b_ref, relu_ref, y_ref):
    i = pl.program_id(0)
    r = jnp.maximum(
        jnp.dot(cur_ref[...], w_ref[...], preferred_element_type=jnp.float32)
        + b_ref[...], 0.0)
    relu_ref[...] = r
    gid = gid_ref[0]
    onehot = (lax.broadcasted_iota(jnp.int32, (N_GRAPHS, BLK), 0)
              == gid).astype(jnp.float32)
    part = jnp.dot(onehot, r, preferred_element_type=jnp.float32)

    @pl.when(i == 0)
    def _():
        y_ref[...] = jnp.zeros_like(y_ref)

    acc = y_ref[...] + part
    y_ref[...] = jnp.where(i == NBLK - 1, jnp.maximum(acc, 0.0), acc)


def _tc3(cur, gids3d, W_out, b):
    return pl.pallas_call(
        _tc3_body,
        grid=(NBLK,),
        in_specs=[
            pl.BlockSpec((BLK, LATENT), lambda i: (i, 0)),
            pl.BlockSpec((1, 1, BLK), lambda i: (i, 0, 0)),
            pl.BlockSpec((LATENT, OUT_D), lambda i: (0, 0)),
            pl.BlockSpec((1, OUT_D), lambda i: (0, 0)),
        ],
        out_specs=[
            pl.BlockSpec((BLK, OUT_D), lambda i: (i, 0)),
            pl.BlockSpec((N_GRAPHS, OUT_D), lambda i: (0, 0)),
        ],
        out_shape=[
            jax.ShapeDtypeStruct((N, OUT_D), jnp.float32),
            jax.ShapeDtypeStruct((N_GRAPHS, OUT_D), jnp.float32),
        ],
        compiler_params=pltpu.CompilerParams(
            dimension_semantics=("arbitrary",)),
    )(cur, gids3d, W_out, b)


def kernel(node_feat, edge_feat, edge_index, graph_ids,
           W_n2l, b_n2l, W_e2l, b_e2l, W_conv, b_conv, W_out, b_out):
    src2d = edge_index[0].reshape(E // CHUNK, CHUNK)
    dst2d = edge_index[1].reshape(E // CHUNK, CHUNK)
    eidx2d = jnp.arange(E, dtype=jnp.int32).reshape(E // CHUNK, CHUNK)
    gids3d = graph_ids.reshape(NBLK, 1, BLK)
    zrows = jnp.zeros((ZROWS, LATENT), jnp.float32)

    edge_emb = _tc0(edge_feat, W_e2l, b_e2l.reshape(1, LATENT))
    pool = _gs_kernel(LATENT)(edge_emb, eidx2d, dst2d, zrows)
    msg, cur = _tc1(node_feat, pool, W_n2l, b_n2l.reshape(1, LATENT))
    for _ in range(MAX_LV):
        parts = _gs_kernel(LATENT)(cur, src2d, dst2d, zrows)
        cur = _tc2(parts, msg, W_conv, b_conv.reshape(1, LATENT))
    reluact, y = _tc3(cur, gids3d, W_out, b_out.reshape(1, OUT_D))
    return (reluact, y)


# speedup vs baseline: 2.6878x; 1.0386x over previous
import functools

import jax
import jax.numpy as jnp
from jax import lax
from jax.experimental import pallas as pl
from jax.experimental.pallas import tpu as pltpu
from jax.experimental.pallas import tpu_sc as plsc

N = 10000
E = 320000
D_NODE = 128
D_EDGE = 16
LATENT = 128
OUT_D = 128
N_GRAPHS = 100
MAX_LV = 3

NUM_SC = 2
NUM_TILES = 16
NW = NUM_SC * NUM_TILES

CHUNK = 125
EPT = E // NW
NCHT = EPT // CHUNK
SCHUNK = 40
NSTAGE = NCHT // SCHUNK
ACC_N = N
TROWS = 640
ZROWS = 80


@functools.cache
def _mesh():
    return plsc.VectorSubcoreMesh(core_axis_name="c", subcore_axis_name="s")


def _make_gs_body(width):
    def body(x_hbm, src_hbm, dst_hbm, z_hbm, out_hbm,
             idx_s, idx_d, buf0, buf1, acc, g0, g1):
        c = lax.axis_index("c")
        t = lax.axis_index("s")
        w = c * NUM_TILES + t
        row0 = pl.multiple_of(w * NCHT, 8)

        base = t * TROWS
        nrows = jnp.minimum(N - base, TROWS)

        def zc(k, _):
            r = pl.multiple_of(base + k * ZROWS, 8)
            pltpu.sync_copy(z_hbm, acc.at[pl.ds(r, ZROWS)])
            return 0

        lax.fori_loop(0, nrows // ZROWS, zc, 0)
        plsc.subcore_barrier()

        def stage(st, _):
            srow = pl.multiple_of(row0 + st * SCHUNK, 8)
            pltpu.sync_copy(src_hbm.at[pl.ds(srow, SCHUNK)], idx_s)
            pltpu.sync_copy(dst_hbm.at[pl.ds(srow, SCHUNK)], idx_d)
            pltpu.async_copy(x_hbm.at[idx_s.at[0]], buf0, g0)

            def pair(jj, _):
                j0 = 2 * jj
                j1 = 2 * jj + 1
                pltpu.async_copy(x_hbm.at[idx_s.at[j1]], buf1, g1)
                pltpu.make_async_copy(
                    x_hbm.at[idx_s.at[j0]], buf0, g0).wait()
                pltpu.sync_copy(buf0, acc.at[idx_d.at[j0]], add=True)

                @pl.when(jj + 1 < SCHUNK // 2)
                def _():
                    pltpu.async_copy(x_hbm.at[idx_s.at[j0 + 2]], buf0, g0)

                pltpu.make_async_copy(
                    x_hbm.at[idx_s.at[j1]], buf1, g1).wait()
                pltpu.sync_copy(buf1, acc.at[idx_d.at[j1]], add=True)
                return 0

            lax.fori_loop(0, SCHUNK // 2, pair, 0)
            return 0

        lax.fori_loop(0, NSTAGE, stage, 0)
        plsc.subcore_barrier()

        def co(k, _):
            r = pl.multiple_of(base + k * ZROWS, 8)
            pltpu.sync_copy(acc.at[pl.ds(r, ZROWS)],
                            out_hbm.at[c, pl.ds(r, ZROWS)])
            return 0

        lax.fori_loop(0, nrows // ZROWS, co, 0)

    return body


@functools.cache
def _gs_kernel(width):
    return pl.kernel(
        _make_gs_body(width),
        out_type=jax.ShapeDtypeStruct((NUM_SC, N, width), jnp.float32),
        name=f"sc_segsum_{width}",
        mesh=_mesh(),
        scratch_types=[
            pltpu.VMEM((SCHUNK, CHUNK), jnp.int32),
            pltpu.VMEM((SCHUNK, CHUNK), jnp.int32),
            pltpu.VMEM((CHUNK, width), jnp.float32),
            pltpu.VMEM((CHUNK, width), jnp.float32),
            pltpu.VMEM_SHARED((ACC_N, width), jnp.float32),
            pltpu.SemaphoreType.DMA,
            pltpu.SemaphoreType.DMA,
        ],
    )


BLK = 1000
NBLK = N // BLK


EBLK = 8000
NEBLK = E // EBLK


def _tc0_body(ef_ref, we_ref, b_ref, emb_ref):
    emb_ref[...] = (
        jnp.dot(ef_ref[...], we_ref[...], preferred_element_type=jnp.float32)
        + b_ref[...])


def _tc0(edge_feat, W_e2l, b):
    return pl.pallas_call(
        _tc0_body,
        grid=(NEBLK,),
        in_specs=[
            pl.BlockSpec((EBLK, D_EDGE), lambda i: (i, 0)),
            pl.BlockSpec((D_EDGE, LATENT), lambda i: (0, 0)),
            pl.BlockSpec((1, LATENT), lambda i: (0, 0)),
        ],
        out_specs=pl.BlockSpec((EBLK, LATENT), lambda i: (i, 0)),
        out_shape=jax.ShapeDtypeStruct((E, LATENT), jnp.float32),
        compiler_params=pltpu.CompilerParams(
            dimension_semantics=("parallel",)),
    )(edge_feat, W_e2l, b)


def _tc1_body(nf_ref, p_ref, wn_ref, b_ref, msg_ref, cur_ref):
    p = p_ref[0] + p_ref[1]
    m = (jnp.dot(nf_ref[...], wn_ref[...], preferred_element_type=jnp.float32)
         + p + b_ref[...])
    msg_ref[...] = m
    cur_ref[...] = jnp.maximum(m, 0.0)


def _tc1(node_feat, pool, W_n2l, b):
    return pl.pallas_call(
        _tc1_body,
        grid=(NBLK,),
        in_specs=[
            pl.BlockSpec((BLK, D_NODE), lambda i: (i, 0)),
            pl.BlockSpec((NUM_SC, BLK, LATENT), lambda i: (0, i, 0)),
            pl.BlockSpec((D_NODE, LATENT), lambda i: (0, 0)),
            pl.BlockSpec((1, LATENT), lambda i: (0, 0)),
        ],
        out_specs=[
            pl.BlockSpec((BLK, LATENT), lambda i: (i, 0)),
            pl.BlockSpec((BLK, LATENT), lambda i: (i, 0)),
        ],
        out_shape=[
            jax.ShapeDtypeStruct((N, LATENT), jnp.float32),
            jax.ShapeDtypeStruct((N, LATENT), jnp.float32),
        ],
        compiler_params=pltpu.CompilerParams(
            dimension_semantics=("parallel",)),
    )(node_feat, pool, W_n2l, b)


def _tc2_body(p_ref, msg_ref, w_ref, b_ref, cur_ref):
    p = p_ref[0] + p_ref[1]
    m = (jnp.dot(p, w_ref[...], preferred_element_type=jnp.float32)
         + b_ref[...] + msg_ref[...])
    cur_ref[...] = jnp.maximum(m, 0.0)


def _tc2(parts, msg, W_conv, b):
    return pl.pallas_call(
        _tc2_body,
        grid=(NBLK,),
        in_specs=[
            pl.BlockSpec((NUM_SC, BLK, LATENT), lambda i: (0, i, 0)),
            pl.BlockSpec((BLK, LATENT), lambda i: (i, 0)),
            pl.BlockSpec((LATENT, LATENT), lambda i: (0, 0)),
            pl.BlockSpec((1, LATENT), lambda i: (0, 0)),
        ],
        out_specs=pl.BlockSpec((BLK, LATENT), lambda i: (i, 0)),
        out_shape=jax.ShapeDtypeStruct((N, LATENT), jnp.float32),
        compiler_params=pltpu.CompilerParams(
            dimension_semantics=("parallel",)),
    )(parts, msg, W_conv, b)


def _tc3_body(cur_ref, gid_ref, w_ref, b_ref, relu_ref, y_ref):
    i = pl.program_id(0)
    r = jnp.maximum(
        jnp.dot(cur_ref[...], w_ref[...], preferred_element_type=jnp.float32)
        + b_ref[...], 0.0)
    relu_ref[...] = r
    gid = gid_ref[0]
    onehot = (lax.broadcasted_iota(jnp.int32, (N_GRAPHS, BLK), 0)
              == gid).astype(jnp.float32)
    part = jnp.dot(onehot, r, preferred_element_type=jnp.float32)

    @pl.when(i == 0)
    def _():
        y_ref[...] = jnp.zeros_like(y_ref)

    acc = y_ref[...] + part
    y_ref[...] = jnp.where(i == NBLK - 1, jnp.maximum(acc, 0.0), acc)


def _tc3(cur, gids3d, W_out, b):
    return pl.pallas_call(
        _tc3_body,
        grid=(NBLK,),
        in_specs=[
            pl.BlockSpec((BLK, LATENT), lambda i: (i, 0)),
            pl.BlockSpec((1, 1, BLK), lambda i: (i, 0, 0)),
            pl.BlockSpec((LATENT, OUT_D), lambda i: (0, 0)),
            pl.BlockSpec((1, OUT_D), lambda i: (0, 0)),
        ],
        out_specs=[
            pl.BlockSpec((BLK, OUT_D), lambda i: (i, 0)),
            pl.BlockSpec((N_GRAPHS, OUT_D), lambda i: (0, 0)),
        ],
        out_shape=[
            jax.ShapeDtypeStruct((N, OUT_D), jnp.float32),
            jax.ShapeDtypeStruct((N_GRAPHS, OUT_D), jnp.float32),
        ],
        compiler_params=pltpu.CompilerParams(
            dimension_semantics=("arbitrary",)),
    )(cur, gids3d, W_out, b)


def kernel(node_feat, edge_feat, edge_index, graph_ids,
           W_n2l, b_n2l, W_e2l, b_e2l, W_conv, b_conv, W_out, b_out):
    src2d = edge_index[0].reshape(E // CHUNK, CHUNK)
    dst2d = edge_index[1].reshape(E // CHUNK, CHUNK)
    eidx2d = jnp.arange(E, dtype=jnp.int32).reshape(E // CHUNK, CHUNK)
    gids3d = graph_ids.reshape(NBLK, 1, BLK)
    zrows = jnp.zeros((ZROWS, LATENT), jnp.float32)

    edge_emb = _tc0(edge_feat, W_e2l, b_e2l.reshape(1, LATENT))
    pool = _gs_kernel(LATENT)(edge_emb, eidx2d, dst2d, zrows)
    msg, cur = _tc1(node_feat, pool, W_n2l, b_n2l.reshape(1, LATENT))
    for _ in range(MAX_LV):
        parts = _gs_kernel(LATENT)(cur, src2d, dst2d, zrows)
        cur = _tc2(parts, msg, W_conv, b_conv.reshape(1, LATENT))
    reluact, y = _tc3(cur, gids3d, W_out, b_out.reshape(1, OUT_D))
    return (reluact, y)
